# Initial kernel scaffold; baseline (speedup 1.0000x reference)
import jax
import jax.numpy as jnp
from jax.experimental import pallas as pl


def kernel(x, kernel_basis, fiber_kernel_basis, edge_index, W_kernel, W_fiber, bias):
    raise NotImplementedError("write your pallas kernel here")


# speedup vs baseline: 9.2476x; 9.2476x over previous
import functools

import jax
import jax.numpy as jnp
from jax import lax
from jax.experimental import pallas as pl
from jax.experimental.pallas import tpu as pltpu
from jax.experimental.pallas import tpu_sc as plsc

N = 10000
E = 160000
O = 8
C = 128
BASIS = 16

NTILES = 16
EPT = E // NTILES
W = 80
NWIN = EPT // W
O_PER_SC = O // 2
STRIPE = 640
LAST_STRIPE = N - 15 * STRIPE


def _kmsg_body(kb_ref, w_ref, out_ref):
    kb = kb_ref[...]
    w = w_ref[...]
    for o in range(O):
        out_ref[o] = jnp.dot(kb[:, o, :], w,
                             preferred_element_type=jnp.float32)


def _kmsg_tc(kernel_basis, W_kernel):
    EB = 2000
    return pl.pallas_call(
        _kmsg_body,
        grid=(E // EB,),
        in_specs=[
            pl.BlockSpec((EB, O, BASIS), lambda i: (i, 0, 0)),
            pl.BlockSpec((BASIS, C), lambda i: (0, 0)),
        ],
        out_specs=pl.BlockSpec((O, EB, C), lambda i: (0, i, 0)),
        out_shape=jax.ShapeDtypeStruct((O, E, C), jnp.float32),
    )(kernel_basis, W_kernel)


def _sc_edge_body(xv_hbm, kmsg_hbm, src_hbm, dst_hbm, zeros_hbm, out_hbm,
                  sbuf0, dbuf0, ibuf0, xbuf0, kbuf0,
                  sbuf1, dbuf1, ibuf1, xbuf1, kbuf1,
                  acc, gsem0, gsem1, ssem0, ssem1):
    cid = lax.axis_index("c")
    sub = lax.axis_index("s")
    base = sub * EPT
    rows0 = sub * STRIPE

    def load_and_gather(w, o, sbuf, dbuf, ibuf, xbuf, kbuf, gsem):
        eb = base + w * W
        pltpu.sync_copy(src_hbm.at[pl.ds(eb, W)], sbuf)
        pltpu.sync_copy(dst_hbm.at[pl.ds(eb, W)], dbuf)
        for ch in range(W // 16):
            sl = pl.ds(ch * 16, 16)
            ibuf[sl] = sbuf[sl] * O + o
        gx = pltpu.async_copy(xv_hbm.at[ibuf], xbuf, gsem)
        gk = pltpu.async_copy(kmsg_hbm.at[o, pl.ds(eb, W)], kbuf, gsem)
        return gx, gk

    def mult_and_scatter(gx, gk, dbuf, xbuf, kbuf, ssem):
        gx.wait()
        gk.wait()

        def mrow(r, carry):
            for ch in range(C // 16):
                sl = pl.ds(ch * 16, 16)
                xbuf[r, sl] = xbuf[r, sl] * kbuf[r, sl]
            return carry

        lax.fori_loop(0, W, mrow, 0)
        return pltpu.async_copy(xbuf, acc.at[dbuf], ssem, add=True)

    for j in range(O_PER_SC):
        o = cid * O_PER_SC + j

        @pl.when(sub < NTILES - 1)
        def _():
            pltpu.sync_copy(zeros_hbm.at[pl.ds(rows0, STRIPE)],
                            acc.at[pl.ds(rows0, STRIPE)])

        @pl.when(sub == NTILES - 1)
        def _():
            pltpu.sync_copy(zeros_hbm.at[pl.ds(rows0, LAST_STRIPE)],
                            acc.at[pl.ds(rows0, LAST_STRIPE)])

        plsc.subcore_barrier()

        def win_pair(i, carry):
            g0 = load_and_gather(2 * i, o, sbuf0, dbuf0, ibuf0, xbuf0, kbuf0,
                                 gsem0)
            g1 = load_and_gather(2 * i + 1, o, sbuf1, dbuf1, ibuf1, xbuf1,
                                 kbuf1, gsem1)
            s0 = mult_and_scatter(*g0, dbuf0, xbuf0, kbuf0, ssem0)
            s1 = mult_and_scatter(*g1, dbuf1, xbuf1, kbuf1, ssem1)
            s0.wait()
            s1.wait()
            return carry

        lax.fori_loop(0, NWIN // 2, win_pair, 0)
        if NWIN % 2:
            g0 = load_and_gather(NWIN - 1, o, sbuf0, dbuf0, ibuf0, xbuf0,
                                 kbuf0, gsem0)
            s0 = mult_and_scatter(*g0, dbuf0, xbuf0, kbuf0, ssem0)
            s0.wait()
        plsc.subcore_barrier()

        @pl.when(sub < NTILES - 1)
        def _():
            pltpu.sync_copy(acc.at[pl.ds(rows0, STRIPE)],
                            out_hbm.at[o, pl.ds(rows0, STRIPE)])

        @pl.when(sub == NTILES - 1)
        def _():
            pltpu.sync_copy(acc.at[pl.ds(rows0, LAST_STRIPE)],
                            out_hbm.at[o, pl.ds(rows0, LAST_STRIPE)])

        plsc.subcore_barrier()


def _sc_edge(xv, kmsg, src, dst, zeros):
    mesh = plsc.VectorSubcoreMesh(core_axis_name="c", subcore_axis_name="s")
    f = functools.partial(
        pl.kernel,
        out_type=jax.ShapeDtypeStruct((O, N, C), jnp.float32),
        mesh=mesh,
        scratch_types=[
            pltpu.VMEM((W,), jnp.int32),
            pltpu.VMEM((W,), jnp.int32),
            pltpu.VMEM((W,), jnp.int32),
            pltpu.VMEM((W, C), jnp.float32),
            pltpu.VMEM((W, C), jnp.float32),
            pltpu.VMEM((W,), jnp.int32),
            pltpu.VMEM((W,), jnp.int32),
            pltpu.VMEM((W,), jnp.int32),
            pltpu.VMEM((W, C), jnp.float32),
            pltpu.VMEM((W, C), jnp.float32),
            pltpu.VMEM_SHARED((N, C), jnp.float32),
            pltpu.SemaphoreType.DMA,
            pltpu.SemaphoreType.DMA,
            pltpu.SemaphoreType.DMA,
            pltpu.SemaphoreType.DMA,
        ],
    )(_sc_edge_body)
    return f(xv, kmsg, src, dst, zeros)


def _fiber_body(x1_ref, fkb_ref, wf_ref, out_ref):
    fk = jnp.dot(fkb_ref[...].reshape(O * O, BASIS), wf_ref[...],
                 preferred_element_type=jnp.float32).reshape(O, O, C)
    fk = fk * (1.0 / O)
    x1 = x1_ref[...]
    for p in range(O):
        acc = x1[0] * fk[p, 0, :][None, :]
        for oo in range(1, O):
            acc = acc + x1[oo] * fk[p, oo, :][None, :]
        out_ref[:, p, :] = acc


def _fiber_tc(x1, fiber_kernel_basis, W_fiber):
    NB = 1000
    return pl.pallas_call(
        _fiber_body,
        grid=(N // NB,),
        in_specs=[
            pl.BlockSpec((O, NB, C), lambda i: (0, i, 0)),
            pl.BlockSpec((O, O, BASIS), lambda i: (0, 0, 0)),
            pl.BlockSpec((BASIS, C), lambda i: (0, 0)),
        ],
        out_specs=pl.BlockSpec((NB, O, C), lambda i: (i, 0, 0)),
        out_shape=jax.ShapeDtypeStruct((N, O, C), jnp.float32),
    )(x1, fiber_kernel_basis, W_fiber)


def kernel(x, kernel_basis, fiber_kernel_basis, edge_index, W_kernel, W_fiber,
           bias):
    del bias
    ei = edge_index.astype(jnp.int32)
    src = ei[0]
    dst = ei[1]
    kmsg = _kmsg_tc(kernel_basis, W_kernel)
    xv = x.reshape(N * O, C)
    zeros = jnp.zeros((N, C), jnp.float32)
    x1 = _sc_edge(xv, kmsg, src, dst, zeros)
    return _fiber_tc(x1, fiber_kernel_basis, W_fiber)


# speedup vs baseline: 10.4484x; 1.1298x over previous
import functools

import jax
import jax.numpy as jnp
from jax import lax
from jax.experimental import pallas as pl
from jax.experimental.pallas import tpu as pltpu
from jax.experimental.pallas import tpu_sc as plsc

N = 10000
E = 160000
O = 8
C = 128
BASIS = 16

NTILES = 16
EPT = E // NTILES
W = 80
NWIN = EPT // W
PAIRS = (NWIN - 3) // 2
O_PER_SC = O // 2
STRIPE = 640
LAST_STRIPE = N - 15 * STRIPE
CH = C // 2


def _kmsg_body(kb_ref, w_ref, out_ref):
    kb = kb_ref[...]
    w = w_ref[...]
    for o in range(O):
        k = jnp.dot(kb[:, o, :], w, preferred_element_type=jnp.float32)
        k32 = lax.bitcast_convert_type(k, jnp.int32)
        lo = lax.shift_right_logical(k32[:, :CH] + 0x8000, 16)
        hi = (k32[:, CH:] + 0x8000) & jnp.int32(-65536)
        out_ref[o] = hi | lo


def _kmsg_tc(kernel_basis, W_kernel):
    EB = 2000
    return pl.pallas_call(
        _kmsg_body,
        grid=(E // EB,),
        in_specs=[
            pl.BlockSpec((EB, O, BASIS), lambda i: (i, 0, 0)),
            pl.BlockSpec((BASIS, C), lambda i: (0, 0)),
        ],
        out_specs=pl.BlockSpec((O, EB, CH), lambda i: (0, i, 0)),
        out_shape=jax.ShapeDtypeStruct((O, E, CH), jnp.int32),
    )(kernel_basis, W_kernel)


def _sc_edge_body(xv_hbm, kp_hbm, src_hbm, dst_hbm, zeros_hbm, out_hbm,
                  sb0, ib0, db0, xb0, kb0,
                  sb1, ib1, db1, xb1, kb1,
                  acc,
                  gs0, gs1, ss0, ss1):
    cid = lax.axis_index("c")
    sub = lax.axis_index("s")
    base = sub * EPT
    rows0 = sub * STRIPE
    slots = [
        (sb0, ib0, db0, xb0, kb0, gs0, ss0),
        (sb1, ib1, db1, xb1, kb1, gs1, ss1),
    ]

    def drain_g(slot):
        sb, ib, db, xb, kb, gs, ss = slot
        pltpu.make_async_copy(src_hbm.at[pl.ds(0, W)], sb, gs).wait()
        pltpu.make_async_copy(dst_hbm.at[pl.ds(0, W)], db, gs).wait()
        pltpu.make_async_copy(xv_hbm.at[pl.ds(0, W)], xb, gs).wait()
        pltpu.make_async_copy(kp_hbm.at[0, pl.ds(0, W)], kb, gs).wait()

    def drain_s(slot):
        sb, ib, db, xb, kb, gs, ss = slot
        pltpu.make_async_copy(xv_hbm.at[pl.ds(0, W)], xb, ss).wait()

    def mult(slot):
        sb, ib, db, xb, kb, gs, ss = slot

        def mrow(r, carry):
            for t in range(4):
                kv = kb[r, pl.ds(t * 16, 16)]
                clo = lax.bitcast_convert_type(lax.shift_left(kv, 16),
                                               jnp.float32)
                chi = lax.bitcast_convert_type(kv & jnp.int32(-65536),
                                               jnp.float32)
                sl = pl.ds(t * 16, 16)
                sh = pl.ds((t + 4) * 16, 16)
                xb[r, sl] = xb[r, sl] * clo
                xb[r, sh] = xb[r, sh] * chi
            return carry

        lax.fori_loop(0, W, mrow, 0)

    def scat(slot):
        sb, ib, db, xb, kb, gs, ss = slot
        pltpu.async_copy(xb, acc.at[db], ss, add=True)

    for j in range(O_PER_SC):
        o = cid * O_PER_SC + j

        def fire(w, slot):
            sb, ib, db, xb, kb, gs, ss = slot
            for ch in range(W // 16):
                ib[pl.ds(ch * 16, 16)] = sb[pl.ds(ch * 16, 16)] * O + o
            pltpu.async_copy(src_hbm.at[pl.ds(base + (w + 2) * W, W)], sb, gs)
            pltpu.async_copy(dst_hbm.at[pl.ds(base + w * W, W)], db, gs)
            pltpu.async_copy(xv_hbm.at[ib], xb, gs)
            pltpu.async_copy(kp_hbm.at[o, pl.ds(base + w * W, W)], kb, gs)

        @pl.when(sub < NTILES - 1)
        def _():
            pltpu.sync_copy(zeros_hbm.at[pl.ds(rows0, STRIPE)],
                            acc.at[pl.ds(rows0, STRIPE)])

        @pl.when(sub == NTILES - 1)
        def _():
            pltpu.sync_copy(zeros_hbm.at[pl.ds(rows0, LAST_STRIPE)],
                            acc.at[pl.ds(rows0, LAST_STRIPE)])

        plsc.subcore_barrier()

        pltpu.sync_copy(src_hbm.at[pl.ds(base, W)], slots[0][0])
        pltpu.sync_copy(src_hbm.at[pl.ds(base + W, W)], slots[1][0])
        fire(0, slots[0])
        fire(1, slots[1])

        def step(i, carry):
            w = 2 * i
            drain_g(slots[0])
            mult(slots[0])
            scat(slots[0])
            drain_g(slots[1])
            mult(slots[1])
            scat(slots[1])
            drain_s(slots[0])
            fire(w + 2, slots[0])
            drain_s(slots[1])
            fire(w + 3, slots[1])
            return carry

        lax.fori_loop(0, PAIRS, step, 0)

        drain_g(slots[0])
        mult(slots[0])
        scat(slots[0])
        drain_g(slots[1])
        mult(slots[1])
        scat(slots[1])
        drain_s(slots[0])
        fire(NWIN - 1, slots[0])
        drain_g(slots[0])
        mult(slots[0])
        scat(slots[0])
        drain_s(slots[0])
        drain_s(slots[1])

        plsc.subcore_barrier()

        @pl.when(sub < NTILES - 1)
        def _():
            pltpu.sync_copy(acc.at[pl.ds(rows0, STRIPE)],
                            out_hbm.at[o, pl.ds(rows0, STRIPE)])

        @pl.when(sub == NTILES - 1)
        def _():
            pltpu.sync_copy(acc.at[pl.ds(rows0, LAST_STRIPE)],
                            out_hbm.at[o, pl.ds(rows0, LAST_STRIPE)])

        plsc.subcore_barrier()


def _sc_edge(xv, kp, src, dst, zeros):
    mesh = plsc.VectorSubcoreMesh(core_axis_name="c", subcore_axis_name="s")
    slot_scratch = []
    for _ in range(2):
        slot_scratch += [
            pltpu.VMEM((W,), jnp.int32),
            pltpu.VMEM((W,), jnp.int32),
            pltpu.VMEM((W,), jnp.int32),
            pltpu.VMEM((W, C), jnp.float32),
            pltpu.VMEM((W, CH), jnp.int32),
        ]
    f = functools.partial(
        pl.kernel,
        out_type=jax.ShapeDtypeStruct((O, N, C), jnp.float32),
        mesh=mesh,
        scratch_types=(
            slot_scratch
            + [pltpu.VMEM_SHARED((N, C), jnp.float32)]
            + [pltpu.SemaphoreType.DMA] * 4
        ),
    )(_sc_edge_body)
    return f(xv, kp, src, dst, zeros)


def _fiber_body(x1_ref, fkb_ref, wf_ref, out_ref):
    fk = jnp.dot(fkb_ref[...].reshape(O * O, BASIS), wf_ref[...],
                 preferred_element_type=jnp.float32).reshape(O, O, C)
    fk = fk * (1.0 / O)
    x1 = x1_ref[...]
    for p in range(O):
        acc = x1[0] * fk[p, 0, :][None, :]
        for oo in range(1, O):
            acc = acc + x1[oo] * fk[p, oo, :][None, :]
        out_ref[:, p, :] = acc


def _fiber_tc(x1, fiber_kernel_basis, W_fiber):
    NB = 1000
    return pl.pallas_call(
        _fiber_body,
        grid=(N // NB,),
        in_specs=[
            pl.BlockSpec((O, NB, C), lambda i: (0, i, 0)),
            pl.BlockSpec((O, O, BASIS), lambda i: (0, 0, 0)),
            pl.BlockSpec((BASIS, C), lambda i: (0, 0)),
        ],
        out_specs=pl.BlockSpec((NB, O, C), lambda i: (i, 0, 0)),
        out_shape=jax.ShapeDtypeStruct((N, O, C), jnp.float32),
    )(x1, fiber_kernel_basis, W_fiber)


def kernel(x, kernel_basis, fiber_kernel_basis, edge_index, W_kernel, W_fiber,
           bias):
    del bias
    ei = edge_index.astype(jnp.int32)
    src = jnp.concatenate([ei[0], jnp.zeros((2 * W,), jnp.int32)])
    dst = ei[1]
    kp = _kmsg_tc(kernel_basis, W_kernel)
    xv = x.reshape(N * O, C)
    zeros = jnp.zeros((N, C), jnp.float32)
    x1 = _sc_edge(xv, kp, src, dst, zeros)
    return _fiber_tc(x1, fiber_kernel_basis, W_fiber)


# speedup vs baseline: 13.4181x; 1.2842x over previous
import functools

import jax
import jax.numpy as jnp
from jax import lax
from jax.experimental import pallas as pl
from jax.experimental.pallas import tpu as pltpu
from jax.experimental.pallas import tpu_sc as plsc

N = 10000
E = 160000
O = 8
C = 128
BASIS = 16

NTILES = 16
EPT = E // NTILES
W = 80
NWIN = EPT // W
PAIRS = (NWIN - 3) // 2
O_PER_SC = O // 2
STRIPE = 640
LAST_STRIPE = N - 15 * STRIPE
CH = C // 2


def _kmsg_body(kb_ref, w_ref, out_ref):
    kb = kb_ref[...].astype(jnp.bfloat16)
    k = jnp.dot(kb, w_ref[...], preferred_element_type=jnp.float32)
    k32 = lax.bitcast_convert_type(k, jnp.int32)
    for o in range(O):
        lo = lax.shift_right_logical(
            k32[:, o * C:o * C + CH] + 0x8000, 16)
        hi = (k32[:, o * C + CH:(o + 1) * C] + 0x8000) & jnp.int32(-65536)
        out_ref[o] = hi | lo


def _kmsg_tc(kernel_basis, W_kernel):
    EB = 2000
    kb2 = kernel_basis.reshape(E, O * BASIS)
    wbd = jnp.kron(jnp.eye(O, dtype=W_kernel.dtype), W_kernel)
    wbd = wbd.astype(jnp.bfloat16)
    return pl.pallas_call(
        _kmsg_body,
        grid=(E // EB,),
        in_specs=[
            pl.BlockSpec((EB, O * BASIS), lambda i: (i, 0)),
            pl.BlockSpec((O * BASIS, O * C), lambda i: (0, 0)),
        ],
        out_specs=pl.BlockSpec((O, EB, CH), lambda i: (0, i, 0)),
        out_shape=jax.ShapeDtypeStruct((O, E, CH), jnp.int32),
    )(kb2, wbd)


def _sc_edge_body(xv_hbm, kp_hbm, src_hbm, dst_hbm, zeros_hbm, out_hbm,
                  sb0, ib0, db0, xb0, kb0,
                  sb1, ib1, db1, xb1, kb1,
                  acc,
                  gs0, gs1, ss0, ss1):
    cid = lax.axis_index("c")
    sub = lax.axis_index("s")
    base = sub * EPT
    rows0 = sub * STRIPE
    slots = [
        (sb0, ib0, db0, xb0, kb0, gs0, ss0),
        (sb1, ib1, db1, xb1, kb1, gs1, ss1),
    ]

    def drain_g(slot):
        sb, ib, db, xb, kb, gs, ss = slot
        pltpu.make_async_copy(src_hbm.at[pl.ds(0, W)], sb, gs).wait()
        pltpu.make_async_copy(dst_hbm.at[pl.ds(0, W)], db, gs).wait()
        pltpu.make_async_copy(xv_hbm.at[pl.ds(0, W)], xb, gs).wait()
        pltpu.make_async_copy(kp_hbm.at[0, pl.ds(0, W)], kb, gs).wait()

    def drain_s(slot):
        sb, ib, db, xb, kb, gs, ss = slot
        pltpu.make_async_copy(xv_hbm.at[pl.ds(0, W)], xb, ss).wait()

    def mult(slot):
        sb, ib, db, xb, kb, gs, ss = slot

        def mrow(r, carry):
            for t in range(4):
                kv = kb[r, pl.ds(t * 16, 16)]
                clo = lax.bitcast_convert_type(lax.shift_left(kv, 16),
                                               jnp.float32)
                chi = lax.bitcast_convert_type(kv & jnp.int32(-65536),
                                               jnp.float32)
                sl = pl.ds(t * 16, 16)
                sh = pl.ds((t + 4) * 16, 16)
                xb[r, sl] = xb[r, sl] * clo
                xb[r, sh] = xb[r, sh] * chi
            return carry

        lax.fori_loop(0, W, mrow, 0)

    def scat(slot):
        sb, ib, db, xb, kb, gs, ss = slot
        pltpu.async_copy(xb, acc.at[db], ss, add=True)

    for j in range(O_PER_SC):
        o = cid * O_PER_SC + j

        def fire(w, slot):
            sb, ib, db, xb, kb, gs, ss = slot
            for ch in range(W // 16):
                ib[pl.ds(ch * 16, 16)] = sb[pl.ds(ch * 16, 16)] * O + o
            pltpu.async_copy(src_hbm.at[pl.ds(base + (w + 2) * W, W)], sb, gs)
            pltpu.async_copy(dst_hbm.at[pl.ds(base + w * W, W)], db, gs)
            pltpu.async_copy(xv_hbm.at[ib], xb, gs)
            pltpu.async_copy(kp_hbm.at[o, pl.ds(base + w * W, W)], kb, gs)

        @pl.when(sub < NTILES - 1)
        def _():
            pltpu.sync_copy(zeros_hbm.at[pl.ds(rows0, STRIPE)],
                            acc.at[pl.ds(rows0, STRIPE)])

        @pl.when(sub == NTILES - 1)
        def _():
            pltpu.sync_copy(zeros_hbm.at[pl.ds(rows0, LAST_STRIPE)],
                            acc.at[pl.ds(rows0, LAST_STRIPE)])

        plsc.subcore_barrier()

        pltpu.sync_copy(src_hbm.at[pl.ds(base, W)], slots[0][0])
        pltpu.sync_copy(src_hbm.at[pl.ds(base + W, W)], slots[1][0])
        fire(0, slots[0])
        fire(1, slots[1])

        def step(i, carry):
            w = 2 * i
            drain_g(slots[0])
            mult(slots[0])
            scat(slots[0])
            drain_g(slots[1])
            mult(slots[1])
            scat(slots[1])
            drain_s(slots[0])
            fire(w + 2, slots[0])
            drain_s(slots[1])
            fire(w + 3, slots[1])
            return carry

        lax.fori_loop(0, PAIRS, step, 0)

        drain_g(slots[0])
        mult(slots[0])
        scat(slots[0])
        drain_g(slots[1])
        mult(slots[1])
        scat(slots[1])
        drain_s(slots[0])
        fire(NWIN - 1, slots[0])
        drain_g(slots[0])
        mult(slots[0])
        scat(slots[0])
        drain_s(slots[0])
        drain_s(slots[1])

        plsc.subcore_barrier()

        @pl.when(sub < NTILES - 1)
        def _():
            pltpu.sync_copy(acc.at[pl.ds(rows0, STRIPE)],
                            out_hbm.at[o, pl.ds(rows0, STRIPE)])

        @pl.when(sub == NTILES - 1)
        def _():
            pltpu.sync_copy(acc.at[pl.ds(rows0, LAST_STRIPE)],
                            out_hbm.at[o, pl.ds(rows0, LAST_STRIPE)])

        plsc.subcore_barrier()


def _sc_edge(xv, kp, src, dst, zeros):
    mesh = plsc.VectorSubcoreMesh(core_axis_name="c", subcore_axis_name="s")
    slot_scratch = []
    for _ in range(2):
        slot_scratch += [
            pltpu.VMEM((W,), jnp.int32),
            pltpu.VMEM((W,), jnp.int32),
            pltpu.VMEM((W,), jnp.int32),
            pltpu.VMEM((W, C), jnp.float32),
            pltpu.VMEM((W, CH), jnp.int32),
        ]
    f = functools.partial(
        pl.kernel,
        out_type=jax.ShapeDtypeStruct((O, N, C), jnp.float32),
        mesh=mesh,
        scratch_types=(
            slot_scratch
            + [pltpu.VMEM_SHARED((N, C), jnp.float32)]
            + [pltpu.SemaphoreType.DMA] * 4
        ),
    )(_sc_edge_body)
    return f(xv, kp, src, dst, zeros)


def _fiber_body(x1_ref, fkb_ref, wf_ref, out_ref):
    fk = jnp.dot(fkb_ref[...].reshape(O * O, BASIS), wf_ref[...],
                 preferred_element_type=jnp.float32).reshape(O, O, C)
    fk = fk * (1.0 / O)
    x1 = x1_ref[...]
    for p in range(O):
        acc = x1[0] * fk[p, 0, :][None, :]
        for oo in range(1, O):
            acc = acc + x1[oo] * fk[p, oo, :][None, :]
        out_ref[:, p, :] = acc


def _fiber_tc(x1, fiber_kernel_basis, W_fiber):
    NB = 1000
    return pl.pallas_call(
        _fiber_body,
        grid=(N // NB,),
        in_specs=[
            pl.BlockSpec((O, NB, C), lambda i: (0, i, 0)),
            pl.BlockSpec((O, O, BASIS), lambda i: (0, 0, 0)),
            pl.BlockSpec((BASIS, C), lambda i: (0, 0)),
        ],
        out_specs=pl.BlockSpec((NB, O, C), lambda i: (i, 0, 0)),
        out_shape=jax.ShapeDtypeStruct((N, O, C), jnp.float32),
    )(x1, fiber_kernel_basis, W_fiber)


def kernel(x, kernel_basis, fiber_kernel_basis, edge_index, W_kernel, W_fiber,
           bias):
    del bias
    ei = edge_index.astype(jnp.int32)
    src = jnp.concatenate([ei[0], jnp.zeros((2 * W,), jnp.int32)])
    dst = ei[1]
    kp = _kmsg_tc(kernel_basis, W_kernel)
    xv = x.reshape(N * O, C)
    zeros = jnp.zeros((N, C), jnp.float32)
    x1 = _sc_edge(xv, kp, src, dst, zeros)
    return _fiber_tc(x1, fiber_kernel_basis, W_fiber)


# speedup vs baseline: 13.4920x; 1.0055x over previous
import functools

import jax
import jax.numpy as jnp
from jax import lax
from jax.experimental import pallas as pl
from jax.experimental.pallas import tpu as pltpu
from jax.experimental.pallas import tpu_sc as plsc

N = 10000
E = 160000
O = 8
C = 128
BASIS = 16

NTILES = 16
EPT = E // NTILES
W = 80
NWIN = EPT // W
PAIRS = (NWIN - 3) // 2
O_PER_SC = O // 2
STRIPE = 640
LAST_STRIPE = N - 15 * STRIPE
CH = C // 2


def _kmsg_body(kb_ref, w_ref, out_ref):
    kb = kb_ref[...].astype(jnp.bfloat16)
    k = jnp.dot(kb, w_ref[...], preferred_element_type=jnp.float32)
    k32 = lax.bitcast_convert_type(k, jnp.int32)
    for o in range(O):
        lo = lax.shift_right_logical(k32[:, o * C:o * C + CH], 16)
        hi = k32[:, o * C + CH:(o + 1) * C] & jnp.int32(-65536)
        out_ref[o] = hi | lo


def _kmsg_tc(kernel_basis, W_kernel):
    EB = 4000
    kb2 = kernel_basis.reshape(E, O * BASIS)
    wbd = jnp.kron(jnp.eye(O, dtype=W_kernel.dtype), W_kernel)
    wbd = wbd.astype(jnp.bfloat16)
    return pl.pallas_call(
        _kmsg_body,
        grid=(E // EB,),
        in_specs=[
            pl.BlockSpec((EB, O * BASIS), lambda i: (i, 0)),
            pl.BlockSpec((O * BASIS, O * C), lambda i: (0, 0)),
        ],
        out_specs=pl.BlockSpec((O, EB, CH), lambda i: (0, i, 0)),
        out_shape=jax.ShapeDtypeStruct((O, E, CH), jnp.int32),
    )(kb2, wbd)


def _sc_edge_body(xv_hbm, kp_hbm, src_hbm, dst_hbm, zeros_hbm, out_hbm,
                  sb0, ib0, db0, xb0, kb0,
                  sb1, ib1, db1, xb1, kb1,
                  acc,
                  gs0, gs1, ss0, ss1):
    cid = lax.axis_index("c")
    sub = lax.axis_index("s")
    base = sub * EPT
    rows0 = sub * STRIPE
    slots = [
        (sb0, ib0, db0, xb0, kb0, gs0, ss0),
        (sb1, ib1, db1, xb1, kb1, gs1, ss1),
    ]

    def drain_g(slot):
        sb, ib, db, xb, kb, gs, ss = slot
        pltpu.make_async_copy(src_hbm.at[pl.ds(0, W)], sb, gs).wait()
        pltpu.make_async_copy(dst_hbm.at[pl.ds(0, W)], db, gs).wait()
        pltpu.make_async_copy(xv_hbm.at[pl.ds(0, W)], xb, gs).wait()
        pltpu.make_async_copy(kp_hbm.at[0, pl.ds(0, W)], kb, gs).wait()

    def drain_s(slot):
        sb, ib, db, xb, kb, gs, ss = slot
        pltpu.make_async_copy(xv_hbm.at[pl.ds(0, W)], xb, ss).wait()

    def mult(slot):
        sb, ib, db, xb, kb, gs, ss = slot

        def mrow(r, carry):
            for t in range(4):
                kv = kb[r, pl.ds(t * 16, 16)]
                clo = lax.bitcast_convert_type(lax.shift_left(kv, 16),
                                               jnp.float32)
                chi = lax.bitcast_convert_type(kv & jnp.int32(-65536),
                                               jnp.float32)
                sl = pl.ds(t * 16, 16)
                sh = pl.ds((t + 4) * 16, 16)
                xb[r, sl] = xb[r, sl] * clo
                xb[r, sh] = xb[r, sh] * chi
            return carry

        lax.fori_loop(0, W, mrow, 0)

    def scat(slot):
        sb, ib, db, xb, kb, gs, ss = slot
        pltpu.async_copy(xb, acc.at[db], ss, add=True)

    for j in range(O_PER_SC):
        o = cid * O_PER_SC + j

        def fire(w, slot):
            sb, ib, db, xb, kb, gs, ss = slot
            for ch in range(W // 16):
                ib[pl.ds(ch * 16, 16)] = sb[pl.ds(ch * 16, 16)] * O + o
            pltpu.async_copy(src_hbm.at[pl.ds(base + (w + 2) * W, W)], sb, gs)
            pltpu.async_copy(dst_hbm.at[pl.ds(base + w * W, W)], db, gs)
            pltpu.async_copy(xv_hbm.at[ib], xb, gs)
            pltpu.async_copy(kp_hbm.at[o, pl.ds(base + w * W, W)], kb, gs)

        @pl.when(sub < NTILES - 1)
        def _():
            pltpu.sync_copy(zeros_hbm.at[pl.ds(rows0, STRIPE)],
                            acc.at[pl.ds(rows0, STRIPE)])

        @pl.when(sub == NTILES - 1)
        def _():
            pltpu.sync_copy(zeros_hbm.at[pl.ds(rows0, LAST_STRIPE)],
                            acc.at[pl.ds(rows0, LAST_STRIPE)])

        plsc.subcore_barrier()

        pltpu.sync_copy(src_hbm.at[pl.ds(base, W)], slots[0][0])
        pltpu.sync_copy(src_hbm.at[pl.ds(base + W, W)], slots[1][0])
        fire(0, slots[0])
        fire(1, slots[1])

        def step(i, carry):
            w = 2 * i
            drain_g(slots[0])
            mult(slots[0])
            scat(slots[0])
            drain_g(slots[1])
            mult(slots[1])
            scat(slots[1])
            drain_s(slots[0])
            fire(w + 2, slots[0])
            drain_s(slots[1])
            fire(w + 3, slots[1])
            return carry

        lax.fori_loop(0, PAIRS, step, 0)

        drain_g(slots[0])
        mult(slots[0])
        scat(slots[0])
        drain_g(slots[1])
        mult(slots[1])
        scat(slots[1])
        drain_s(slots[0])
        fire(NWIN - 1, slots[0])
        drain_g(slots[0])
        mult(slots[0])
        scat(slots[0])
        drain_s(slots[0])
        drain_s(slots[1])

        plsc.subcore_barrier()

        @pl.when(sub < NTILES - 1)
        def _():
            pltpu.sync_copy(acc.at[pl.ds(rows0, STRIPE)],
                            out_hbm.at[o, pl.ds(rows0, STRIPE)])

        @pl.when(sub == NTILES - 1)
        def _():
            pltpu.sync_copy(acc.at[pl.ds(rows0, LAST_STRIPE)],
                            out_hbm.at[o, pl.ds(rows0, LAST_STRIPE)])

        plsc.subcore_barrier()


def _sc_edge(xv, kp, src, dst, zeros):
    mesh = plsc.VectorSubcoreMesh(core_axis_name="c", subcore_axis_name="s")
    slot_scratch = []
    for _ in range(2):
        slot_scratch += [
            pltpu.VMEM((W,), jnp.int32),
            pltpu.VMEM((W,), jnp.int32),
            pltpu.VMEM((W,), jnp.int32),
            pltpu.VMEM((W, C), jnp.float32),
            pltpu.VMEM((W, CH), jnp.int32),
        ]
    f = functools.partial(
        pl.kernel,
        out_type=jax.ShapeDtypeStruct((O, N, C), jnp.float32),
        mesh=mesh,
        scratch_types=(
            slot_scratch
            + [pltpu.VMEM_SHARED((N, C), jnp.float32)]
            + [pltpu.SemaphoreType.DMA] * 4
        ),
    )(_sc_edge_body)
    return f(xv, kp, src, dst, zeros)


def _fiber_body(x1_ref, fkb_ref, wf_ref, out_ref):
    fk = jnp.dot(fkb_ref[...].reshape(O * O, BASIS), wf_ref[...],
                 preferred_element_type=jnp.float32).reshape(O, O, C)
    fk = fk * (1.0 / O)
    x1 = x1_ref[...]
    for p in range(O):
        acc = x1[0] * fk[p, 0, :][None, :]
        for oo in range(1, O):
            acc = acc + x1[oo] * fk[p, oo, :][None, :]
        out_ref[:, p, :] = acc


def _fiber_tc(x1, fiber_kernel_basis, W_fiber):
    NB = 1000
    return pl.pallas_call(
        _fiber_body,
        grid=(N // NB,),
        in_specs=[
            pl.BlockSpec((O, NB, C), lambda i: (0, i, 0)),
            pl.BlockSpec((O, O, BASIS), lambda i: (0, 0, 0)),
            pl.BlockSpec((BASIS, C), lambda i: (0, 0)),
        ],
        out_specs=pl.BlockSpec((NB, O, C), lambda i: (i, 0, 0)),
        out_shape=jax.ShapeDtypeStruct((N, O, C), jnp.float32),
    )(x1, fiber_kernel_basis, W_fiber)


def kernel(x, kernel_basis, fiber_kernel_basis, edge_index, W_kernel, W_fiber,
           bias):
    del bias
    ei = edge_index.astype(jnp.int32)
    src = jnp.concatenate([ei[0], jnp.zeros((2 * W,), jnp.int32)])
    dst = ei[1]
    kp = _kmsg_tc(kernel_basis, W_kernel)
    xv = x.reshape(N * O, C)
    zeros = jnp.zeros((N, C), jnp.float32)
    x1 = _sc_edge(xv, kp, src, dst, zeros)
    return _fiber_tc(x1, fiber_kernel_basis, W_fiber)
